# baseline (speedup 1.0000x reference)
import functools

import jax
import jax.numpy as jnp
from jax import lax
from jax.experimental import pallas as pl
from jax.experimental.pallas import tpu as pltpu
from jax.experimental.pallas import tpu_sc as plsc

N_TOK = 9216
N_CODES = 8192
D = 256

BZ = 2304
SB = 512
NS = N_CODES // SB
CH = 128
NCH = SB // CH


def _merge(av, aa, bv, ba):
    better = bv < av
    return jnp.where(better, bv, av), jnp.where(better, ba, aa)


def _argmin_body(z_ref, emb_ref, idx_ref, et_ref, en_ref):
    i = pl.program_id(0)

    @pl.when(i == 0)
    def _():
        for s in range(NS):
            et_ref[:, s * SB:(s + 1) * SB] = emb_ref[s * SB:(s + 1) * SB, :].T
        et = et_ref[...]
        en_ref[...] = jnp.sum(et * et, axis=0, keepdims=True)

    z = z_ref[...]
    zn = jnp.sum(z * z, axis=1, keepdims=True)
    zm2 = -(z + z)

    accv = acca = None
    for s in range(NS):
        m2 = lax.dot_general(
            zm2, et_ref[:, s * SB:(s + 1) * SB],
            (((1,), (0,)), ((), ())), preferred_element_type=jnp.float32)
        sv = sa = None
        for c in range(NCH):
            col0 = s * SB + c * CH
            dv = (zn + en_ref[:, col0:col0 + CH]) + m2[:, c * CH:(c + 1) * CH]
            da = jnp.full((BZ, CH), float(col0), jnp.float32)
            if sv is None:
                sv, sa = dv, da
            else:
                sv, sa = _merge(sv, sa, dv, da)
        if accv is None:
            accv, acca = sv, sa
        else:
            accv, acca = _merge(accv, acca, sv, sa)

    gm = jnp.min(accv, axis=1, keepdims=True)
    lanef = lax.broadcasted_iota(jnp.int32, (BZ, CH), 1).astype(jnp.float32)
    cand = jnp.where(accv == gm, acca + lanef, 3e38)
    idx_ref[...] = jnp.min(cand, axis=1, keepdims=True).astype(jnp.int32)


def _argmin_call(z, embedding):
    ntok = z.shape[0]
    grid = (ntok // BZ,)
    return pl.pallas_call(
        _argmin_body,
        grid=grid,
        in_specs=[
            pl.BlockSpec((BZ, D), lambda i: (i, 0)),
            pl.BlockSpec((N_CODES, D), lambda i: (0, 0)),
        ],
        out_specs=pl.BlockSpec((BZ, 1), lambda i: (i, 0)),
        out_shape=jax.ShapeDtypeStruct((ntok, 1), jnp.int32),
        scratch_shapes=[
            pltpu.VMEM((D, N_CODES), jnp.float32),
            pltpu.VMEM((1, N_CODES), jnp.float32),
        ],
        compiler_params=pltpu.CompilerParams(
            dimension_semantics=("arbitrary",),
        ),
    )(z, embedding)


_NW = 32


def _gather_call(embedding, idx):
    ntok = idx.shape[0]
    bpw = ntok // _NW
    mesh = plsc.VectorSubcoreMesh(core_axis_name="c", subcore_axis_name="s")

    @functools.partial(
        pl.kernel,
        mesh=mesh,
        out_type=jax.ShapeDtypeStruct((ntok, D), jnp.float32),
        scratch_types=[
            pltpu.VMEM((bpw,), jnp.int32),
            pltpu.VMEM((bpw, D), jnp.float32),
            pltpu.SemaphoreType.DMA,
        ],
    )
    def k(table_hbm, idx_hbm, out_hbm, idx_v, rows_v, sem):
        wid = lax.axis_index("s") * 2 + lax.axis_index("c")
        base = wid * bpw
        pltpu.sync_copy(idx_hbm.at[pl.ds(base, bpw)], idx_v)
        pltpu.async_copy(table_hbm.at[idx_v], rows_v, sem).wait()
        pltpu.sync_copy(rows_v, out_hbm.at[pl.ds(base, bpw)])

    return k(embedding, idx)


def kernel(z, embedding):
    idx = _argmin_call(z, embedding).reshape(N_TOK)
    z_q = _gather_call(embedding, idx)
    return z_q, idx


# speedup vs baseline: 1.0223x; 1.0223x over previous
import functools

import jax
import jax.numpy as jnp
from jax import lax
from jax.experimental import pallas as pl
from jax.experimental.pallas import tpu as pltpu
from jax.experimental.pallas import tpu_sc as plsc

N_TOK = 9216
N_CODES = 8192
D = 256

BZ = 3072
SB = 1024
NS = N_CODES // SB
CH = 128
NCH = SB // CH


def _merge(av, aa, bv, ba):
    better = bv < av
    return jnp.where(better, bv, av), jnp.where(better, ba, aa)


def _argmin_body(z_ref, emb_ref, idx_ref, et_ref, en_ref):
    i = pl.program_id(0)

    @pl.when(i == 0)
    def _():
        for s in range(NS):
            et_ref[:, s * SB:(s + 1) * SB] = emb_ref[s * SB:(s + 1) * SB, :].T
        et = et_ref[...]
        en_ref[...] = jnp.sum(et * et, axis=0, keepdims=True)

    z = z_ref[...]
    zn = jnp.sum(z * z, axis=1, keepdims=True)
    zm2 = -(z + z)

    accv = acca = None
    for s in range(NS):
        m2 = lax.dot_general(
            zm2, et_ref[:, s * SB:(s + 1) * SB],
            (((1,), (0,)), ((), ())), preferred_element_type=jnp.float32)
        sv = sa = None
        for c in range(NCH):
            col0 = s * SB + c * CH
            dv = (zn + en_ref[:, col0:col0 + CH]) + m2[:, c * CH:(c + 1) * CH]
            da = jnp.full((BZ, CH), float(col0), jnp.float32)
            if sv is None:
                sv, sa = dv, da
            else:
                sv, sa = _merge(sv, sa, dv, da)
        if accv is None:
            accv, acca = sv, sa
        else:
            accv, acca = _merge(accv, acca, sv, sa)

    gm = jnp.min(accv, axis=1, keepdims=True)
    lanef = lax.broadcasted_iota(jnp.int32, (BZ, CH), 1).astype(jnp.float32)
    cand = jnp.where(accv == gm, acca + lanef, 3e38)
    idx_ref[...] = jnp.min(cand, axis=1, keepdims=True).astype(jnp.int32)


def _argmin_call(z, embedding):
    ntok = z.shape[0]
    grid = (ntok // BZ,)
    return pl.pallas_call(
        _argmin_body,
        grid=grid,
        in_specs=[
            pl.BlockSpec((BZ, D), lambda i: (i, 0)),
            pl.BlockSpec((N_CODES, D), lambda i: (0, 0)),
        ],
        out_specs=pl.BlockSpec((BZ, 1), lambda i: (i, 0)),
        out_shape=jax.ShapeDtypeStruct((ntok, 1), jnp.int32),
        scratch_shapes=[
            pltpu.VMEM((D, N_CODES), jnp.float32),
            pltpu.VMEM((1, N_CODES), jnp.float32),
        ],
        compiler_params=pltpu.CompilerParams(
            dimension_semantics=("arbitrary",),
        ),
    )(z, embedding)


_NW = 32


def _gather_call(embedding, idx):
    ntok = idx.shape[0]
    bpw = ntok // _NW
    mesh = plsc.VectorSubcoreMesh(core_axis_name="c", subcore_axis_name="s")

    @functools.partial(
        pl.kernel,
        mesh=mesh,
        out_type=jax.ShapeDtypeStruct((ntok, D), jnp.float32),
        scratch_types=[
            pltpu.VMEM((bpw,), jnp.int32),
            pltpu.VMEM((bpw, D), jnp.float32),
            pltpu.SemaphoreType.DMA,
        ],
    )
    def k(table_hbm, idx_hbm, out_hbm, idx_v, rows_v, sem):
        wid = lax.axis_index("s") * 2 + lax.axis_index("c")
        base = wid * bpw
        pltpu.sync_copy(idx_hbm.at[pl.ds(base, bpw)], idx_v)
        pltpu.async_copy(table_hbm.at[idx_v], rows_v, sem).wait()
        pltpu.sync_copy(rows_v, out_hbm.at[pl.ds(base, bpw)])

    return k(embedding, idx)


def kernel(z, embedding):
    idx = _argmin_call(z, embedding).reshape(N_TOK)
    z_q = _gather_call(embedding, idx)
    return z_q, idx


# speedup vs baseline: 1.0360x; 1.0135x over previous
import functools

import jax
import jax.numpy as jnp
from jax import lax
from jax.experimental import pallas as pl
from jax.experimental.pallas import tpu as pltpu
from jax.experimental.pallas import tpu_sc as plsc

N_TOK = 9216
N_CODES = 8192
D = 256

BZ = 4608
SB = 512
NS = N_CODES // SB
CH = 128
NCH = SB // CH


def _merge(av, aa, bv, ba):
    better = bv < av
    return jnp.where(better, bv, av), jnp.where(better, ba, aa)


def _argmin_body(z_ref, emb_ref, idx_ref, et_ref, en_ref):
    i = pl.program_id(0)

    @pl.when(i == 0)
    def _():
        for s in range(NS):
            et_ref[:, s * SB:(s + 1) * SB] = emb_ref[s * SB:(s + 1) * SB, :].T
        et = et_ref[...]
        en_ref[...] = jnp.sum(et * et, axis=0, keepdims=True)

    z = z_ref[...]
    zn = jnp.sum(z * z, axis=1, keepdims=True)
    zm2 = -(z + z)

    accv = acca = None
    for s in range(NS):
        m2 = lax.dot_general(
            zm2, et_ref[:, s * SB:(s + 1) * SB],
            (((1,), (0,)), ((), ())), preferred_element_type=jnp.float32)
        sv = sa = None
        for c in range(NCH):
            col0 = s * SB + c * CH
            dv = (zn + en_ref[:, col0:col0 + CH]) + m2[:, c * CH:(c + 1) * CH]
            da = jnp.full((BZ, CH), float(col0), jnp.float32)
            if sv is None:
                sv, sa = dv, da
            else:
                sv, sa = _merge(sv, sa, dv, da)
        if accv is None:
            accv, acca = sv, sa
        else:
            accv, acca = _merge(accv, acca, sv, sa)

    gm = jnp.min(accv, axis=1, keepdims=True)
    lanef = lax.broadcasted_iota(jnp.int32, (BZ, CH), 1).astype(jnp.float32)
    cand = jnp.where(accv == gm, acca + lanef, 3e38)
    idx_ref[...] = jnp.min(cand, axis=1, keepdims=True).astype(jnp.int32)


def _argmin_call(z, embedding):
    ntok = z.shape[0]
    grid = (ntok // BZ,)
    return pl.pallas_call(
        _argmin_body,
        grid=grid,
        in_specs=[
            pl.BlockSpec((BZ, D), lambda i: (i, 0)),
            pl.BlockSpec((N_CODES, D), lambda i: (0, 0)),
        ],
        out_specs=pl.BlockSpec((BZ, 1), lambda i: (i, 0)),
        out_shape=jax.ShapeDtypeStruct((ntok, 1), jnp.int32),
        scratch_shapes=[
            pltpu.VMEM((D, N_CODES), jnp.float32),
            pltpu.VMEM((1, N_CODES), jnp.float32),
        ],
        compiler_params=pltpu.CompilerParams(
            dimension_semantics=("arbitrary",),
        ),
    )(z, embedding)


_NW = 32


def _gather_call(embedding, idx):
    ntok = idx.shape[0]
    bpw = ntok // _NW
    mesh = plsc.VectorSubcoreMesh(core_axis_name="c", subcore_axis_name="s")

    @functools.partial(
        pl.kernel,
        mesh=mesh,
        out_type=jax.ShapeDtypeStruct((ntok, D), jnp.float32),
        scratch_types=[
            pltpu.VMEM((bpw,), jnp.int32),
            pltpu.VMEM((bpw, D), jnp.float32),
            pltpu.SemaphoreType.DMA,
        ],
    )
    def k(table_hbm, idx_hbm, out_hbm, idx_v, rows_v, sem):
        wid = lax.axis_index("s") * 2 + lax.axis_index("c")
        base = wid * bpw
        pltpu.sync_copy(idx_hbm.at[pl.ds(base, bpw)], idx_v)
        pltpu.async_copy(table_hbm.at[idx_v], rows_v, sem).wait()
        pltpu.sync_copy(rows_v, out_hbm.at[pl.ds(base, bpw)])

    return k(embedding, idx)


def kernel(z, embedding):
    idx = _argmin_call(z, embedding).reshape(N_TOK)
    z_q = _gather_call(embedding, idx)
    return z_q, idx
